# baseline (speedup 1.0000x reference)
import functools

import jax
import jax.numpy as jnp
from jax import lax
from jax.experimental import pallas as pl
from jax.experimental.pallas import tpu as pltpu
from jax.experimental.pallas import tpu_sc as plsc

NUM_FACTORS = 16
BATCH = 16384

_NC, _NS = 2, 16
_NW = _NC * _NS
_BPW = BATCH // _NW
_GROUP = 16


def _mf_body(user_hbm, item_hbm, uf_hbm, if_hbm, out_hbm,
             uidx_v, iidx_v, urows_v, vrows_v, out_v, sem_u, sem_v):
    wid = lax.axis_index("s") * _NC + lax.axis_index("c")
    base = wid * _BPW

    pltpu.sync_copy(user_hbm.at[pl.ds(base, _BPW)], uidx_v)
    pltpu.sync_copy(item_hbm.at[pl.ds(base, _BPW)], iidx_v)

    cp_u = pltpu.async_copy(uf_hbm.at[uidx_v], urows_v, sem_u)
    cp_v = pltpu.async_copy(if_hbm.at[iidx_v], vrows_v, sem_v)
    cp_u.wait()
    cp_v.wait()

    lane = lax.iota(jnp.int32, 16)
    diags = [(lane + k) & 15 for k in range(16)]

    def step(g, carry):
        r0 = g * _GROUP
        rows = r0 + lane
        acc = jnp.zeros((16,), jnp.float32)
        for k in range(16):
            du = plsc.load_gather(urows_v, [rows, diags[k]])
            dv = plsc.load_gather(vrows_v, [rows, diags[k]])
            acc = acc + du * dv
        out_v[pl.ds(r0, _GROUP)] = acc
        return carry

    lax.fori_loop(0, _BPW // _GROUP, step, 0, unroll=False)

    pltpu.sync_copy(out_v, out_hbm.at[pl.ds(base, _BPW)])


@jax.jit
def _mf_call(user, item, user_factors, item_factors):
    mesh = plsc.VectorSubcoreMesh(
        core_axis_name="c", subcore_axis_name="s",
        num_cores=_NC, num_subcores=_NS)
    return pl.kernel(
        _mf_body,
        out_type=jax.ShapeDtypeStruct((BATCH,), jnp.float32),
        mesh=mesh,
        compiler_params=pltpu.CompilerParams(
            needs_layout_passes=False, use_tc_tiling_on_sc=False),
        scratch_types=[
            pltpu.VMEM((_BPW,), jnp.int32),
            pltpu.VMEM((_BPW,), jnp.int32),
            pltpu.VMEM((_BPW, NUM_FACTORS), jnp.float32),
            pltpu.VMEM((_BPW, NUM_FACTORS), jnp.float32),
            pltpu.VMEM((_BPW,), jnp.float32),
            pltpu.SemaphoreType.DMA,
            pltpu.SemaphoreType.DMA,
        ],
    )(user, item, user_factors, item_factors)


def kernel(user, item, user_factors, item_factors):
    user = user.astype(jnp.int32)
    item = item.astype(jnp.int32)
    return _mf_call(user, item, user_factors, item_factors)
